# Initial kernel scaffold; baseline (speedup 1.0000x reference)
import functools

import jax
import jax.numpy as jnp
from jax import lax
from jax.experimental import pallas as pl
from jax.experimental.pallas import tpu as pltpu
from jax.experimental.pallas import tpu_sc as plsc

NUM_ENTITIES = 100000
NUM_RELATIONS = 1000
D = 128
B = 16384
L = 16
NC = 2
NS = 16
NW = NC * NS
B_PER_W = B // NW
CHUNK = 128
NCHUNK = B_PER_W // CHUNK


def _rsqrt(x):
    i = lax.bitcast_convert_type(x, jnp.int32)
    i = jnp.int32(0x5F3759DF) - lax.shift_right_logical(i, 1)
    y = lax.bitcast_convert_type(i, jnp.float32)
    xh = x * jnp.float32(0.5)
    for _ in range(3):
        y = y * (jnp.float32(1.5) - xh * y * y)
    return y


def _body(ent_hbm, rel_hbm, hid_hbm, rid_hbm, tid_hbm, out_hbm,
          hidx, ridx, tidx, hrows, rrows, trows, outv, sem):
    wid = lax.axis_index("s") * NC + lax.axis_index("c")
    base = wid * B_PER_W

    for c in range(NCHUNK):
        off = base + c * CHUNK
        pltpu.sync_copy(hid_hbm.at[pl.ds(off, CHUNK)], hidx)
        pltpu.sync_copy(rid_hbm.at[pl.ds(off, CHUNK)], ridx)
        pltpu.sync_copy(tid_hbm.at[pl.ds(off, CHUNK)], tidx)
        cph = pltpu.async_copy(ent_hbm.at[hidx], hrows, sem)
        cpr = pltpu.async_copy(rel_hbm.at[ridx], rrows, sem)
        cpt = pltpu.async_copy(ent_hbm.at[tidx], trows, sem)
        cph.wait()
        cpr.wait()
        cpt.wait()

        def row(i, carry):
            hv = [hrows[i, pl.ds(j * L, L)] for j in range(D // L)]
            tv = [trows[i, pl.ds(j * L, L)] for j in range(D // L)]
            hh = hv[0] * hv[0]
            tt = tv[0] * tv[0]
            for j in range(1, D // L):
                hh = hh + hv[j] * hv[j]
                tt = tt + tv[j] * tv[j]
            s = _rsqrt(jnp.sum(hh))
            u = _rsqrt(jnp.sum(tt))
            dd = None
            for j in range(D // L):
                d = hv[j] * s + rrows[i, pl.ds(j * L, L)] - tv[j] * u
                dd = d * d if dd is None else dd + d * d
            dds = jnp.sum(dd)
            outv[c * CHUNK + i] = -(dds * _rsqrt(dds))
            return carry

        lax.fori_loop(0, CHUNK, row, 0)

    pltpu.sync_copy(outv, out_hbm.at[pl.ds(base, B_PER_W)])


@functools.partial(
    pl.kernel,
    out_type=jax.ShapeDtypeStruct((B,), jnp.float32),
    mesh=plsc.VectorSubcoreMesh(core_axis_name="c", subcore_axis_name="s"),
    scratch_types=[
        pltpu.VMEM((CHUNK,), jnp.int32),
        pltpu.VMEM((CHUNK,), jnp.int32),
        pltpu.VMEM((CHUNK,), jnp.int32),
        pltpu.VMEM((CHUNK, D), jnp.float32),
        pltpu.VMEM((CHUNK, D), jnp.float32),
        pltpu.VMEM((CHUNK, D), jnp.float32),
        pltpu.VMEM((B_PER_W,), jnp.float32),
        pltpu.SemaphoreType.DMA,
    ],
)
def _sc_kernel(*refs):
    _body(*refs)


def kernel(entity_emb, relation_emb, head_ids, relation_ids, tail_ids):
    return _sc_kernel(
        entity_emb,
        relation_emb,
        head_ids.astype(jnp.int32),
        relation_ids.astype(jnp.int32),
        tail_ids.astype(jnp.int32),
    )


# speedup vs baseline: 1.3497x; 1.3497x over previous
import functools

import jax
import jax.numpy as jnp
from jax import lax
from jax.experimental import pallas as pl
from jax.experimental.pallas import tpu as pltpu
from jax.experimental.pallas import tpu_sc as plsc

NUM_ENTITIES = 100000
NUM_RELATIONS = 1000
D = 128
B = 16384
L = 16
NC = 2
NS = 16
NW = NC * NS
B_PER_W = B // NW
CHUNK = 128
NCHUNK = B_PER_W // CHUNK


def _rsqrt(x):
    i = lax.bitcast_convert_type(x, jnp.int32)
    i = jnp.int32(0x5F3759DF) - lax.shift_right_logical(i, 1)
    y = lax.bitcast_convert_type(i, jnp.float32)
    xh = x * jnp.float32(0.5)
    for _ in range(3):
        y = y * (jnp.float32(1.5) - xh * y * y)
    return y


def _body(ent_hbm, rel_hbm, hid_hbm, rid_hbm, tid_hbm, out_hbm,
          hidx, ridx, tidx, hrows, rrows, trows, outv, sem):
    wid = lax.axis_index("s") * NC + lax.axis_index("c")
    base = wid * B_PER_W

    for c in range(NCHUNK):
        off = base + c * CHUNK
        pltpu.sync_copy(hid_hbm.at[pl.ds(off, CHUNK)], hidx)
        pltpu.sync_copy(rid_hbm.at[pl.ds(off, CHUNK)], ridx)
        pltpu.sync_copy(tid_hbm.at[pl.ds(off, CHUNK)], tidx)
        cph = pltpu.async_copy(ent_hbm.at[hidx], hrows, sem)
        cpr = pltpu.async_copy(rel_hbm.at[ridx], rrows, sem)
        cpt = pltpu.async_copy(ent_hbm.at[tidx], trows, sem)
        cph.wait()
        cpr.wait()
        cpt.wait()

        def row(i, carry):
            hv = [hrows[i, pl.ds(j * L, L)] for j in range(D // L)]
            tv = [trows[i, pl.ds(j * L, L)] for j in range(D // L)]
            hh = hv[0] * hv[0]
            tt = tv[0] * tv[0]
            for j in range(1, D // L):
                hh = hh + hv[j] * hv[j]
                tt = tt + tv[j] * tv[j]
            s = _rsqrt(jnp.sum(hh))
            u = _rsqrt(jnp.sum(tt))
            dd = None
            for j in range(D // L):
                d = hv[j] * s + rrows[i, pl.ds(j * L, L)] - tv[j] * u
                dd = d * d if dd is None else dd + d * d
            dds = jnp.sum(dd)
            score = jnp.full((L,), -(dds * _rsqrt(dds)), jnp.float32)
            lane0 = lax.iota(jnp.int32, L) == 0
            plsc.store_scatter(
                outv, [jnp.full((L,), c * CHUNK + i, jnp.int32)], score,
                mask=lane0)
            return carry

        lax.fori_loop(0, CHUNK, row, 0)

    pltpu.sync_copy(outv, out_hbm.at[pl.ds(base, B_PER_W)])


@functools.partial(
    pl.kernel,
    out_type=jax.ShapeDtypeStruct((B,), jnp.float32),
    mesh=plsc.VectorSubcoreMesh(core_axis_name="c", subcore_axis_name="s"),
    compiler_params=pltpu.CompilerParams(needs_layout_passes=False),
    scratch_types=[
        pltpu.VMEM((CHUNK,), jnp.int32),
        pltpu.VMEM((CHUNK,), jnp.int32),
        pltpu.VMEM((CHUNK,), jnp.int32),
        pltpu.VMEM((CHUNK, D), jnp.float32),
        pltpu.VMEM((CHUNK, D), jnp.float32),
        pltpu.VMEM((CHUNK, D), jnp.float32),
        pltpu.VMEM((B_PER_W,), jnp.float32),
        pltpu.SemaphoreType.DMA,
    ],
)
def _sc_kernel(*refs):
    _body(*refs)


def kernel(entity_emb, relation_emb, head_ids, relation_ids, tail_ids):
    return _sc_kernel(
        entity_emb,
        relation_emb,
        head_ids.astype(jnp.int32),
        relation_ids.astype(jnp.int32),
        tail_ids.astype(jnp.int32),
    )
